# baseline (speedup 1.0000x reference)
import jax
import jax.numpy as jnp
from jax.experimental import pallas as pl
from jax.experimental.pallas import tpu as pltpu

B = 16384
D = 64

_NSLAB = 16
_RS = B // _NSLAB


def _tc_body(gu_hbm, gi_hbm, out_ref, gu_v, gi_v, sems):
    for s in range(_NSLAB):
        pltpu.make_async_copy(
            gu_hbm.at[pl.ds(s * _RS, _RS), :],
            gu_v.at[pl.ds(s * _RS, _RS), :],
            sems.at[0, s]).start()
        pltpu.make_async_copy(
            gi_hbm.at[pl.ds(s * _RS, _RS), :],
            gi_v.at[pl.ds(s * _RS, _RS), :],
            sems.at[1, s]).start()
    for s in range(_NSLAB):
        pltpu.make_async_copy(
            gu_hbm.at[pl.ds(s * _RS, _RS), :],
            gu_v.at[pl.ds(s * _RS, _RS), :],
            sems.at[0, s]).wait()
        pltpu.make_async_copy(
            gi_hbm.at[pl.ds(s * _RS, _RS), :],
            gi_v.at[pl.ds(s * _RS, _RS), :],
            sems.at[1, s]).wait()
        rows = pl.ds(s * _RS, _RS)
        out_ref[rows] = jnp.sum(gu_v[rows, :] * gi_v[rows, :], axis=1)


@jax.jit
def _tc_rowdot(gu, gi):
    return pl.pallas_call(
        _tc_body,
        in_specs=[
            pl.BlockSpec(memory_space=pl.ANY),
            pl.BlockSpec(memory_space=pl.ANY),
        ],
        out_shape=jax.ShapeDtypeStruct((B,), jnp.float32),
        scratch_shapes=[
            pltpu.VMEM((B, D), jnp.float32),
            pltpu.VMEM((B, D), jnp.float32),
            pltpu.SemaphoreType.DMA((2, _NSLAB)),
        ],
    )(gu, gi)


def kernel(gu, gi):
    return _tc_rowdot(jnp.squeeze(gu), jnp.squeeze(gi))


# speedup vs baseline: 1.0201x; 1.0201x over previous
import jax
import jax.numpy as jnp
from jax.experimental import pallas as pl
from jax.experimental.pallas import tpu as pltpu

B = 16384
D = 64

_NSLAB = 4
_RS = B // _NSLAB


def _tc_body(gu_hbm, gi_hbm, out_ref, gu_v, gi_v, sems):
    for s in range(_NSLAB):
        pltpu.make_async_copy(
            gu_hbm.at[pl.ds(s * _RS, _RS), :],
            gu_v.at[pl.ds(s * _RS, _RS), :],
            sems.at[0, s]).start()
        pltpu.make_async_copy(
            gi_hbm.at[pl.ds(s * _RS, _RS), :],
            gi_v.at[pl.ds(s * _RS, _RS), :],
            sems.at[1, s]).start()
    for s in range(_NSLAB):
        pltpu.make_async_copy(
            gu_hbm.at[pl.ds(s * _RS, _RS), :],
            gu_v.at[pl.ds(s * _RS, _RS), :],
            sems.at[0, s]).wait()
        pltpu.make_async_copy(
            gi_hbm.at[pl.ds(s * _RS, _RS), :],
            gi_v.at[pl.ds(s * _RS, _RS), :],
            sems.at[1, s]).wait()
        rows = pl.ds(s * _RS, _RS)
        out_ref[rows] = jnp.sum(gu_v[rows, :] * gi_v[rows, :], axis=1)


@jax.jit
def _tc_rowdot(gu, gi):
    return pl.pallas_call(
        _tc_body,
        in_specs=[
            pl.BlockSpec(memory_space=pl.ANY),
            pl.BlockSpec(memory_space=pl.ANY),
        ],
        out_shape=jax.ShapeDtypeStruct((B,), jnp.float32),
        scratch_shapes=[
            pltpu.VMEM((B, D), jnp.float32),
            pltpu.VMEM((B, D), jnp.float32),
            pltpu.SemaphoreType.DMA((2, _NSLAB)),
        ],
    )(gu, gi)


def kernel(gu, gi):
    return _tc_rowdot(jnp.squeeze(gu), jnp.squeeze(gi))


# speedup vs baseline: 1.0325x; 1.0122x over previous
import jax
import jax.numpy as jnp
from jax.experimental import pallas as pl
from jax.experimental.pallas import tpu as pltpu

B = 16384
D = 64

_NSLAB = 8
_RS = B // _NSLAB


def _tc_body(gu_hbm, gi_hbm, out_ref, gu_v, gi_v, sems):
    for s in range(_NSLAB):
        pltpu.make_async_copy(
            gu_hbm.at[pl.ds(s * _RS, _RS), :],
            gu_v.at[pl.ds(s * _RS, _RS), :],
            sems.at[0, s]).start()
        pltpu.make_async_copy(
            gi_hbm.at[pl.ds(s * _RS, _RS), :],
            gi_v.at[pl.ds(s * _RS, _RS), :],
            sems.at[1, s]).start()
    for s in range(_NSLAB):
        pltpu.make_async_copy(
            gu_hbm.at[pl.ds(s * _RS, _RS), :],
            gu_v.at[pl.ds(s * _RS, _RS), :],
            sems.at[0, s]).wait()
        pltpu.make_async_copy(
            gi_hbm.at[pl.ds(s * _RS, _RS), :],
            gi_v.at[pl.ds(s * _RS, _RS), :],
            sems.at[1, s]).wait()
        rows = pl.ds(s * _RS, _RS)
        out_ref[rows] = jnp.sum(gu_v[rows, :] * gi_v[rows, :], axis=1)


@jax.jit
def _tc_rowdot(gu, gi):
    return pl.pallas_call(
        _tc_body,
        in_specs=[
            pl.BlockSpec(memory_space=pl.ANY),
            pl.BlockSpec(memory_space=pl.ANY),
        ],
        out_shape=jax.ShapeDtypeStruct((B,), jnp.float32),
        scratch_shapes=[
            pltpu.VMEM((B, D), jnp.float32),
            pltpu.VMEM((B, D), jnp.float32),
            pltpu.SemaphoreType.DMA((2, _NSLAB)),
        ],
    )(gu, gi)


def kernel(gu, gi):
    return _tc_rowdot(jnp.squeeze(gu), jnp.squeeze(gi))
